# Initial kernel scaffold; baseline (speedup 1.0000x reference)
import jax
import jax.numpy as jnp
from jax.experimental import pallas as pl


def kernel(rules, relation, rules_weight, bias):
    raise NotImplementedError("write your pallas kernel here")


# speedup vs baseline: 128.4572x; 128.4572x over previous
import jax
import jax.numpy as jnp
from jax import lax
from jax.experimental import pallas as pl
from jax.experimental.pallas import tpu as pltpu
from jax.experimental.pallas import tpu_sc as plsc

B = 16384
L = 200
NUM_W = 1000001
NUM_REL = 1000

NC, NS, LANES = 2, 16, 16
NW = NC * NS
ROWS_PER_W = B // NW
CHUNK = 64
NCHUNK = ROWS_PER_W // CHUNK
CW = CHUNK * L
FULL_VREGS = L // LANES
TAIL = L - FULL_VREGS * LANES


def _body(rules_hbm, rel_hbm, w_hbm, bias_hbm, out_hbm,
          rules_v, vals_v, bias_v, rel_v, out_acc, sem):
  wid = lax.axis_index("s") * NC + lax.axis_index("c")
  wbase = wid * ROWS_PER_W

  pltpu.sync_copy(bias_hbm, bias_v)
  pltpu.sync_copy(rel_hbm.at[pl.ds(wbase, ROWS_PER_W)], rel_v)

  row_stride = lax.iota(jnp.int32, LANES) * L

  for c in range(NCHUNK):
    pltpu.sync_copy(rules_hbm.at[pl.ds((wbase + c * CHUNK) * L, CW)], rules_v)
    pltpu.async_copy(w_hbm.at[rules_v], vals_v, sem).wait()

    def group_body(g, carry, c=c):
      base_idx = row_stride + g * (LANES * L)

      def l_body(l, acc):
        return acc + plsc.load_gather(vals_v, [base_idx + l])

      acc = lax.fori_loop(0, L, l_body, jnp.zeros((LANES,), jnp.float32))
      out_acc[pl.ds(c * CHUNK + g * LANES, LANES)] = acc
      return carry

    lax.fori_loop(0, CHUNK // LANES, group_body, 0)

  def bias_body(g, carry):
    idx = rel_v[pl.ds(g * LANES, LANES)]
    out_acc[pl.ds(g * LANES, LANES)] = (
        out_acc[pl.ds(g * LANES, LANES)] + plsc.load_gather(bias_v, [idx]))
    return carry

  lax.fori_loop(0, ROWS_PER_W // LANES, bias_body, 0)

  pltpu.sync_copy(out_acc, out_hbm.at[pl.ds(wbase, ROWS_PER_W)])


@jax.jit
def _run(rules_flat, relation, w_flat, bias_flat):
  mesh = plsc.VectorSubcoreMesh(
      core_axis_name="c", subcore_axis_name="s",
      num_cores=NC, num_subcores=NS)
  f = pl.kernel(
      _body,
      out_type=jax.ShapeDtypeStruct((B,), jnp.float32),
      mesh=mesh,
      compiler_params=pltpu.CompilerParams(needs_layout_passes=False),
      scratch_types=[
          pltpu.VMEM((CW,), jnp.int32),
          pltpu.VMEM((CW,), jnp.float32),
          pltpu.VMEM((NUM_REL,), jnp.float32),
          pltpu.VMEM((ROWS_PER_W,), jnp.int32),
          pltpu.VMEM((ROWS_PER_W,), jnp.float32),
          pltpu.SemaphoreType.DMA,
      ],
  )
  return f(rules_flat, relation, w_flat, bias_flat)


def kernel(rules, relation, rules_weight, bias):
  rules_flat = rules.astype(jnp.int32).reshape(B * L)
  relation = relation.astype(jnp.int32)
  w_flat = rules_weight.reshape(NUM_W)
  bias_flat = bias.reshape(NUM_REL)
  out = _run(rules_flat, relation, w_flat, bias_flat)
  return out.reshape(B, 1)


# speedup vs baseline: 149.0689x; 1.1605x over previous
import jax
import jax.numpy as jnp
from jax import lax
from jax.experimental import pallas as pl
from jax.experimental.pallas import tpu as pltpu
from jax.experimental.pallas import tpu_sc as plsc

B = 16384
L = 200
NUM_W = 1000001
NUM_REL = 1000

NC, NS, LANES = 2, 16, 16
NW = NC * NS
ROWS_PER_W = B // NW
CHUNK = 64
NCHUNK = ROWS_PER_W // CHUNK
CW = CHUNK * L
FULL_VREGS = L // LANES
TAIL = L - FULL_VREGS * LANES


NGROUP = CHUNK // LANES


def _body(rules_hbm, rel_hbm, w_hbm, bias_hbm, out_hbm,
          rules_a, rules_b, vals_a, vals_b, bias_v, rel_v, out_acc,
          rsem, gsem):
  wid = lax.axis_index("s") * NC + lax.axis_index("c")
  wbase = wid * ROWS_PER_W

  pltpu.sync_copy(bias_hbm, bias_v)
  pltpu.sync_copy(rel_hbm.at[pl.ds(wbase, ROWS_PER_W)], rel_v)

  row_stride = lax.iota(jnp.int32, LANES) * L
  base_idx = [row_stride + g * (LANES * L) for g in range(NGROUP)]
  zero = jnp.zeros((LANES,), jnp.float32)

  rules_bufs = [rules_a, rules_b]
  vals_bufs = [vals_a, vals_b]

  def rules_dma(c, buf):
    return pltpu.async_copy(
        rules_hbm.at[pl.ds((wbase + c * CHUNK) * L, CW)], rules_bufs[buf],
        rsem)

  def gather_dma(buf):
    return pltpu.async_copy(w_hbm.at[rules_bufs[buf]], vals_bufs[buf], gsem)

  rules_dma(0, 0).wait()
  g_cur = gather_dma(0)
  r_next = rules_dma(1, 1) if NCHUNK > 1 else None

  for c in range(NCHUNK):
    buf = c % 2
    g_cur.wait()
    if c + 1 < NCHUNK:
      r_next.wait()
      if c + 2 < NCHUNK:
        r_next = rules_dma(c + 2, buf)
      g_cur = gather_dma(1 - buf)

    vals_ref = vals_bufs[buf]

    def l_body(l, accs, vals_ref=vals_ref):
      return tuple(
          accs[g] + plsc.load_gather(vals_ref, [base_idx[g] + l])
          for g in range(NGROUP))

    accs = lax.fori_loop(0, L, l_body, (zero,) * NGROUP, unroll=8)
    for g in range(NGROUP):
      out_acc[pl.ds(c * CHUNK + g * LANES, LANES)] = accs[g]

  def bias_body(g, carry):
    idx = rel_v[pl.ds(g * LANES, LANES)]
    out_acc[pl.ds(g * LANES, LANES)] = (
        out_acc[pl.ds(g * LANES, LANES)] + plsc.load_gather(bias_v, [idx]))
    return carry

  lax.fori_loop(0, ROWS_PER_W // LANES, bias_body, 0)

  pltpu.sync_copy(out_acc, out_hbm.at[pl.ds(wbase, ROWS_PER_W)])


@jax.jit
def _run(rules_flat, relation, w_flat, bias_flat):
  mesh = plsc.VectorSubcoreMesh(
      core_axis_name="c", subcore_axis_name="s",
      num_cores=NC, num_subcores=NS)
  f = pl.kernel(
      _body,
      out_type=jax.ShapeDtypeStruct((B,), jnp.float32),
      mesh=mesh,
      compiler_params=pltpu.CompilerParams(needs_layout_passes=False),
      scratch_types=[
          pltpu.VMEM((CW,), jnp.int32),
          pltpu.VMEM((CW,), jnp.int32),
          pltpu.VMEM((CW,), jnp.float32),
          pltpu.VMEM((CW,), jnp.float32),
          pltpu.VMEM((NUM_REL,), jnp.float32),
          pltpu.VMEM((ROWS_PER_W,), jnp.int32),
          pltpu.VMEM((ROWS_PER_W,), jnp.float32),
          pltpu.SemaphoreType.DMA,
          pltpu.SemaphoreType.DMA,
      ],
  )
  return f(rules_flat, relation, w_flat, bias_flat)


def kernel(rules, relation, rules_weight, bias):
  rules_flat = rules.astype(jnp.int32).reshape(B * L)
  relation = relation.astype(jnp.int32)
  w_flat = rules_weight.reshape(NUM_W)
  bias_flat = bias.reshape(NUM_REL)
  out = _run(rules_flat, relation, w_flat, bias_flat)
  return out.reshape(B, 1)


# speedup vs baseline: 246.4661x; 1.6534x over previous
import jax
import jax.numpy as jnp
from jax import lax
from jax.experimental import pallas as pl
from jax.experimental.pallas import tpu as pltpu
from jax.experimental.pallas import tpu_sc as plsc

B = 16384
L = 200
NUM_W = 1000001
NUM_REL = 1000

NC, NS, LANES = 2, 16, 16
NW = NC * NS
ROWS_PER_W = B // NW
CHUNK = 64
NCHUNK = ROWS_PER_W // CHUNK
CW = CHUNK * L
FULL_VREGS = L // LANES
TAIL = L - FULL_VREGS * LANES


NGROUP = CHUNK // LANES
W_SLICE = 62504
NUM_W_PAD = W_SLICE * NS


def _body(rules_hbm, rel_hbm, w_hbm, bias_hbm, out_hbm,
          rules_a, rules_b, vals_a, vals_b, bias_v, rel_v, out_acc,
          w_spmem, rsem, gsem):
  sid = lax.axis_index("s")
  wid = sid * NC + lax.axis_index("c")
  wbase = wid * ROWS_PER_W

  bounce = [vals_a, vals_b]
  n_full, tail = divmod(W_SLICE, CW)
  for k in range(n_full + 1):
    n = CW if k < n_full else tail
    off = sid * W_SLICE + k * CW
    pltpu.sync_copy(w_hbm.at[pl.ds(off, n)], bounce[k % 2].at[pl.ds(0, n)])
    pltpu.sync_copy(bounce[k % 2].at[pl.ds(0, n)], w_spmem.at[pl.ds(off, n)])
  pltpu.sync_copy(bias_hbm, bias_v)
  pltpu.sync_copy(rel_hbm.at[pl.ds(wbase, ROWS_PER_W)], rel_v)
  plsc.subcore_barrier()

  row_stride = lax.iota(jnp.int32, LANES) * L
  base_idx = [row_stride + g * (LANES * L) for g in range(NGROUP)]
  zero = jnp.zeros((LANES,), jnp.float32)

  rules_bufs = [rules_a, rules_b]
  vals_bufs = [vals_a, vals_b]

  def rules_dma(c, buf):
    return pltpu.async_copy(
        rules_hbm.at[pl.ds((wbase + c * CHUNK) * L, CW)], rules_bufs[buf],
        rsem)

  def gather_dma(buf):
    return pltpu.async_copy(w_spmem.at[rules_bufs[buf]], vals_bufs[buf], gsem)

  rules_dma(0, 0).wait()
  g_cur = gather_dma(0)
  r_next = rules_dma(1, 1) if NCHUNK > 1 else None

  for c in range(NCHUNK):
    buf = c % 2
    g_cur.wait()
    if c + 1 < NCHUNK:
      r_next.wait()
      if c + 2 < NCHUNK:
        r_next = rules_dma(c + 2, buf)
      g_cur = gather_dma(1 - buf)

    vals_ref = vals_bufs[buf]

    def l_body(l, accs, vals_ref=vals_ref):
      return tuple(
          accs[g] + plsc.load_gather(vals_ref, [base_idx[g] + l])
          for g in range(NGROUP))

    accs = lax.fori_loop(0, L, l_body, (zero,) * NGROUP, unroll=8)
    for g in range(NGROUP):
      out_acc[pl.ds(c * CHUNK + g * LANES, LANES)] = accs[g]

  def bias_body(g, carry):
    idx = rel_v[pl.ds(g * LANES, LANES)]
    out_acc[pl.ds(g * LANES, LANES)] = (
        out_acc[pl.ds(g * LANES, LANES)] + plsc.load_gather(bias_v, [idx]))
    return carry

  lax.fori_loop(0, ROWS_PER_W // LANES, bias_body, 0)

  pltpu.sync_copy(out_acc, out_hbm.at[pl.ds(wbase, ROWS_PER_W)])


@jax.jit
def _run(rules_flat, relation, w_flat, bias_flat):
  mesh = plsc.VectorSubcoreMesh(
      core_axis_name="c", subcore_axis_name="s",
      num_cores=NC, num_subcores=NS)
  f = pl.kernel(
      _body,
      out_type=jax.ShapeDtypeStruct((B,), jnp.float32),
      mesh=mesh,
      compiler_params=pltpu.CompilerParams(needs_layout_passes=False),
      scratch_types=[
          pltpu.VMEM((CW,), jnp.int32),
          pltpu.VMEM((CW,), jnp.int32),
          pltpu.VMEM((CW,), jnp.float32),
          pltpu.VMEM((CW,), jnp.float32),
          pltpu.VMEM((NUM_REL,), jnp.float32),
          pltpu.VMEM((ROWS_PER_W,), jnp.int32),
          pltpu.VMEM((ROWS_PER_W,), jnp.float32),
          pltpu.VMEM_SHARED((NUM_W_PAD,), jnp.float32),
          pltpu.SemaphoreType.DMA,
          pltpu.SemaphoreType.DMA,
      ],
  )
  return f(rules_flat, relation, w_flat, bias_flat)


def kernel(rules, relation, rules_weight, bias):
  rules_flat = rules.astype(jnp.int32).reshape(B * L)
  relation = relation.astype(jnp.int32)
  w_flat = jnp.concatenate([
      rules_weight.reshape(NUM_W),
      jnp.zeros((NUM_W_PAD - NUM_W,), jnp.float32)])
  bias_flat = bias.reshape(NUM_REL)
  out = _run(rules_flat, relation, w_flat, bias_flat)
  return out.reshape(B, 1)
